# baseline (speedup 1.0000x reference)
import functools

import jax
import jax.numpy as jnp
from jax import lax
from jax.experimental import pallas as pl
from jax.experimental.pallas import tpu as pltpu
from jax.experimental.pallas import tpu_sc as plsc

_N = 10000
_E = 320000
_D = 256
_CH = 80
_ROWS = _E // _CH
_TILES = 16
_RPT = _ROWS // _TILES
_WPT = 632
_NPAD = _WPT * _TILES

_BN = 1000
_G = _N // _BN


_DH = 128


def _sc_zero_acc(zbuf, acc, base):
    def zrow(i, _):
        def zcol(j, _2):
            zbuf[i, pl.ds(j * 16, 16)] = jnp.zeros((16,), jnp.float32)
            return 0
        return lax.fori_loop(0, _DH // 16, zcol, 0)
    lax.fori_loop(0, _CH, zrow, 0)

    nfull = _WPT // _CH
    rem = _WPT - nfull * _CH
    for q in range(nfull):
        pltpu.sync_copy(zbuf, acc.at[pl.ds(base + q * _CH, _CH)])
    if rem:
        pltpu.sync_copy(zbuf.at[pl.ds(0, rem)],
                        acc.at[pl.ds(base + nfull * _CH, rem)])


def _sc_edge_loop(tab, dst, ebase, sall, didx, rows, acc, sems, nch):
    ds, gs, ss = sems

    def dload(g, b):
        pltpu.async_copy(dst.at[pl.ds(ebase + g * _CH, _CH)],
                         didx[b].at[0], ds[b])

    def gload(g, b):
        pltpu.async_copy(tab.at[sall.at[pl.ds(g * _CH, _CH)]], rows[b], gs[b])

    def dwait(g, b):
        pltpu.make_async_copy(dst.at[pl.ds(ebase + g * _CH, _CH)],
                              didx[b].at[0], ds[b]).wait()

    def gwait(g, b):
        pltpu.make_async_copy(tab.at[sall.at[pl.ds(g * _CH, _CH)]],
                              rows[b], gs[b]).wait()

    def swait(b):
        pltpu.make_async_copy(rows[b], acc.at[didx[b].at[0]], ss[b]).wait()

    for b in (0, 1):
        dload(b, b)
        gload(b, b)

    npairs = nch // 2

    def body(p, _):
        for b in (0, 1):
            g = 2 * p + b
            dwait(g, b)
            gwait(g, b)
            pltpu.async_copy(rows[b], acc.at[didx[b].at[0]], ss[b], add=True)

        @pl.when(p + 1 < npairs)
        def _():
            for b in (0, 1):
                swait(b)
                dload(2 * p + 2 + b, b)
                gload(2 * p + 2 + b, b)
        return 0

    lax.fori_loop(0, npairs, body, 0)
    for b in (0, 1):
        swait(b)
    if nch % 2:
        g = nch - 1
        dload(g, 0)
        gload(g, 0)
        dwait(g, 0)
        gwait(g, 0)
        pltpu.async_copy(rows[0], acc.at[didx[0].at[0]], ss[0], add=True)
        swait(0)


def _sc_writeback(c, base, acc, out0, out1):
    @pl.when(c == 0)
    def _():
        pltpu.sync_copy(acc.at[pl.ds(base, _WPT)], out0.at[pl.ds(base, _WPT)])

    @pl.when(c == 1)
    def _():
        pltpu.sync_copy(acc.at[pl.ds(base, _WPT)], out1.at[pl.ds(base, _WPT)])


def _sc_mesh():
    return plsc.VectorSubcoreMesh(core_axis_name="c", subcore_axis_name="s")


_EPT = _E // _TILES


@functools.cache
def _make_sc_agg_split():
    @functools.partial(
        pl.kernel,
        out_type=(jax.ShapeDtypeStruct((_NPAD, _DH), jnp.float32),
                  jax.ShapeDtypeStruct((_NPAD, _DH), jnp.float32)),
        mesh=_sc_mesh(),
        scratch_types=[
            pltpu.VMEM((_EPT,), jnp.int32),
            pltpu.VMEM((1, _CH), jnp.int32),
            pltpu.VMEM((1, _CH), jnp.int32),
            pltpu.VMEM((_CH, _DH), jnp.float32),
            pltpu.VMEM((_CH, _DH), jnp.float32),
            pltpu.VMEM_SHARED((_NPAD, _DH), jnp.float32),
            pltpu.SemaphoreType.DMA,
            pltpu.SemaphoreType.DMA,
            pltpu.SemaphoreType.DMA,
            pltpu.SemaphoreType.DMA,
            pltpu.SemaphoreType.DMA,
            pltpu.SemaphoreType.DMA,
        ],
    )
    def agg(h0, h1, src, dst, out0, out1, sall, d0, d1, r0, r1, acc,
            ds0, ds1, gs0, gs1, ss0, ss1):
        c = lax.axis_index("c")
        s = lax.axis_index("s")
        base = s * _WPT
        _sc_zero_acc(r0, acc, base)
        plsc.subcore_barrier()

        ebase = s * _EPT
        pltpu.sync_copy(src.at[pl.ds(ebase, _EPT)], sall)
        sems = ((ds0, ds1), (gs0, gs1), (ss0, ss1))

        @pl.when(c == 0)
        def _():
            _sc_edge_loop(h0, dst, ebase, sall, (d0, d1), (r0, r1), acc,
                          sems, _EPT // _CH)

        @pl.when(c == 1)
        def _():
            _sc_edge_loop(h1, dst, ebase, sall, (d0, d1), (r0, r1), acc,
                          sems, _EPT // _CH)

        plsc.subcore_barrier()
        _sc_writeback(c, base, acc, out0, out1)

    return agg


@functools.cache
def _make_sc_agg_full():
    @functools.partial(
        pl.kernel,
        out_type=(jax.ShapeDtypeStruct((_NPAD, _DH), jnp.float32),
                  jax.ShapeDtypeStruct((_NPAD, _DH), jnp.float32)),
        mesh=_sc_mesh(),
        scratch_types=[
            pltpu.VMEM((_EPT // 2,), jnp.int32),
            pltpu.VMEM((1, _CH), jnp.int32),
            pltpu.VMEM((1, _CH), jnp.int32),
            pltpu.VMEM((_CH, _DH), jnp.float32),
            pltpu.VMEM((_CH, _DH), jnp.float32),
            pltpu.VMEM_SHARED((_NPAD, _DH), jnp.float32),
            pltpu.SemaphoreType.DMA,
            pltpu.SemaphoreType.DMA,
            pltpu.SemaphoreType.DMA,
            pltpu.SemaphoreType.DMA,
            pltpu.SemaphoreType.DMA,
            pltpu.SemaphoreType.DMA,
        ],
    )
    def agg(h, src, dst, out0, out1, sall, d0, d1, r0, r1, acc,
            ds0, ds1, gs0, gs1, ss0, ss1):
        c = lax.axis_index("c")
        s = lax.axis_index("s")
        base = s * _WPT
        _sc_zero_acc(r0, acc, base)
        plsc.subcore_barrier()

        ebase = (c * _TILES + s) * (_EPT // 2)
        pltpu.sync_copy(src.at[pl.ds(ebase, _EPT // 2)], sall)
        sems = ((ds0, ds1), (gs0, gs1), (ss0, ss1))

        _sc_edge_loop(h, dst, ebase, sall, (d0, d1), (r0, r1), acc,
                      sems, _EPT // 2 // _CH)

        plsc.subcore_barrier()
        _sc_writeback(c, base, acc, out0, out1)

    return agg


def _lrelu(t):
    return jnp.where(t >= 0, t, 0.01 * t)


def _norm(block, st_ref, vst_ref, g_ref, b_ref):
    mu = st_ref[0:1, :] / _N
    var = vst_ref[0:1, :] / _N
    return (block - mu) / jnp.sqrt(var + 1e-5) * g_ref[...] + b_ref[...]


def _acc_stats(st_ref, v):
    @pl.when(pl.program_id(0) == 0)
    def _():
        st_ref[...] = jnp.zeros_like(st_ref)
    st_ref[0:1, :] += jnp.sum(v, axis=0, keepdims=True)


def _kv_kernel(v_ref, st_ref, vst_ref):
    mu = st_ref[0:1, :] / _N
    dv = v_ref[...] - mu
    _acc_stats(vst_ref, dv * dv)


def _kv_call(v, st):
    return pl.pallas_call(
        _kv_kernel,
        grid=(_G,),
        in_specs=[
            pl.BlockSpec((_BN, _D), lambda i: (i, 0)),
            pl.BlockSpec((8, _D), lambda i: (0, 0)),
        ],
        out_specs=[pl.BlockSpec((8, _D), lambda i: (0, 0))],
        out_shape=[jax.ShapeDtypeStruct((8, _D), jnp.float32)],
    )(v, st)[0]


def _ka_split_kernel(eps_ref, h0_ref, h1_ref, a0_ref, a1_ref, w1_ref, b1_ref,
                     m_ref, st_ref):
    h = jnp.concatenate([h0_ref[...], h1_ref[...]], axis=1)
    a = jnp.concatenate([a0_ref[...], a1_ref[...]], axis=1)
    mm = jnp.dot((1.0 + eps_ref[0]) * h + a, w1_ref[...],
                 preferred_element_type=jnp.float32) + b1_ref[...]
    m_ref[...] = mm
    _acc_stats(st_ref, mm)


def _ka_full_kernel(eps_ref, h_ref, a0_ref, a1_ref, w1_ref, b1_ref,
                    m_ref, st_ref):
    mm = jnp.dot((1.0 + eps_ref[0]) * h_ref[...] + a0_ref[...] + a1_ref[...],
                 w1_ref[...], preferred_element_type=jnp.float32) + b1_ref[...]
    m_ref[...] = mm
    _acc_stats(st_ref, mm)


def _ka_call(eps, hs, a0, a1, w1, b1):
    din = w1.shape[0]
    body = _ka_split_kernel if len(hs) == 2 else _ka_full_kernel
    h_specs = [pl.BlockSpec((_BN, h.shape[1]), lambda i: (i, 0)) for h in hs]
    return pl.pallas_call(
        body,
        grid=(_G,),
        in_specs=[
            pl.BlockSpec(memory_space=pltpu.SMEM),
            *h_specs,
            pl.BlockSpec((_BN, _DH), lambda i: (i, 0)),
            pl.BlockSpec((_BN, _DH), lambda i: (i, 0)),
            pl.BlockSpec((din, _D), lambda i: (0, 0)),
            pl.BlockSpec((1, _D), lambda i: (0, 0)),
        ],
        out_specs=[
            pl.BlockSpec((_BN, _D), lambda i: (i, 0)),
            pl.BlockSpec((8, _D), lambda i: (0, 0)),
        ],
        out_shape=[
            jax.ShapeDtypeStruct((_N, _D), jnp.float32),
            jax.ShapeDtypeStruct((8, _D), jnp.float32),
        ],
    )(eps, *hs, a0, a1, w1, b1)


def _kb_kernel(m_ref, st_ref, vst_ref, g_ref, be_ref, w2_ref, b2_ref,
               h_ref, st2_ref):
    t = _lrelu(_norm(m_ref[...], st_ref, vst_ref, g_ref, be_ref))
    hh = jnp.dot(t, w2_ref[...], preferred_element_type=jnp.float32) + b2_ref[...]
    h_ref[...] = hh
    _acc_stats(st2_ref, hh)


def _kb_call(m, st, vst, g, be, w2, b2):
    return pl.pallas_call(
        _kb_kernel,
        grid=(_G,),
        in_specs=[
            pl.BlockSpec((_BN, _D), lambda i: (i, 0)),
            pl.BlockSpec((8, _D), lambda i: (0, 0)),
            pl.BlockSpec((8, _D), lambda i: (0, 0)),
            pl.BlockSpec((1, _D), lambda i: (0, 0)),
            pl.BlockSpec((1, _D), lambda i: (0, 0)),
            pl.BlockSpec((_D, _D), lambda i: (0, 0)),
            pl.BlockSpec((1, _D), lambda i: (0, 0)),
        ],
        out_specs=[
            pl.BlockSpec((_BN, _D), lambda i: (i, 0)),
            pl.BlockSpec((8, _D), lambda i: (0, 0)),
        ],
        out_shape=[
            jax.ShapeDtypeStruct((_N, _D), jnp.float32),
            jax.ShapeDtypeStruct((8, _D), jnp.float32),
        ],
    )(m, st, vst, g, be, w2, b2)


def _kb2_kernel(m_ref, st_ref, vst_ref, g_ref, be_ref, w2_ref, b2_ref,
                cw1_ref, cb1_ref, c_ref, stc_ref):
    t = _lrelu(_norm(m_ref[...], st_ref, vst_ref, g_ref, be_ref))
    hh = jnp.dot(t, w2_ref[...], preferred_element_type=jnp.float32) + b2_ref[...]
    cc = jnp.dot(hh, cw1_ref[...], preferred_element_type=jnp.float32) + cb1_ref[...]
    c_ref[...] = cc
    _acc_stats(stc_ref, cc)


def _kb2_call(m, st, vst, g, be, w2, b2, cw1, cb1):
    return pl.pallas_call(
        _kb2_kernel,
        grid=(_G,),
        in_specs=[
            pl.BlockSpec((_BN, _D), lambda i: (i, 0)),
            pl.BlockSpec((8, _D), lambda i: (0, 0)),
            pl.BlockSpec((8, _D), lambda i: (0, 0)),
            pl.BlockSpec((1, _D), lambda i: (0, 0)),
            pl.BlockSpec((1, _D), lambda i: (0, 0)),
            pl.BlockSpec((_D, _D), lambda i: (0, 0)),
            pl.BlockSpec((1, _D), lambda i: (0, 0)),
            pl.BlockSpec((_D, _D), lambda i: (0, 0)),
            pl.BlockSpec((1, _D), lambda i: (0, 0)),
        ],
        out_specs=[
            pl.BlockSpec((_BN, _D), lambda i: (i, 0)),
            pl.BlockSpec((8, _D), lambda i: (0, 0)),
        ],
        out_shape=[
            jax.ShapeDtypeStruct((_N, _D), jnp.float32),
            jax.ShapeDtypeStruct((8, _D), jnp.float32),
        ],
    )(m, st, vst, g, be, w2, b2, cw1, cb1)


def _kc_kernel(h_ref, st_ref, vst_ref, g_ref, b_ref, o0_ref, o1_ref):
    t = _lrelu(_norm(h_ref[...], st_ref, vst_ref, g_ref, b_ref))
    o0_ref[...] = t[:, :_D // 2]
    o1_ref[...] = t[:, _D // 2:]


def _kc_call(h, st, vst, g, b):
    dh = _D // 2
    return pl.pallas_call(
        _kc_kernel,
        grid=(_G,),
        in_specs=[
            pl.BlockSpec((_BN, _D), lambda i: (i, 0)),
            pl.BlockSpec((8, _D), lambda i: (0, 0)),
            pl.BlockSpec((8, _D), lambda i: (0, 0)),
            pl.BlockSpec((1, _D), lambda i: (0, 0)),
            pl.BlockSpec((1, _D), lambda i: (0, 0)),
        ],
        out_specs=[
            pl.BlockSpec((_BN, dh), lambda i: (i, 0)),
            pl.BlockSpec((_BN, dh), lambda i: (i, 0)),
        ],
        out_shape=[
            jax.ShapeDtypeStruct((_N, dh), jnp.float32),
            jax.ShapeDtypeStruct((_N, dh), jnp.float32),
        ],
    )(h, st, vst, g, b)


def _ke_kernel(c_ref, st_ref, vst_ref, g_ref, be_ref, w2_ref, b2_ref, o_ref):
    t = _lrelu(_norm(c_ref[...], st_ref, vst_ref, g_ref, be_ref))
    o_ref[...] = jnp.dot(t, w2_ref[...],
                         preferred_element_type=jnp.float32) + b2_ref[...]


def _ke_call(c, st, vst, g, be, w2, b2):
    return pl.pallas_call(
        _ke_kernel,
        grid=(_G,),
        in_specs=[
            pl.BlockSpec((_BN, _D), lambda i: (i, 0)),
            pl.BlockSpec((8, _D), lambda i: (0, 0)),
            pl.BlockSpec((8, _D), lambda i: (0, 0)),
            pl.BlockSpec((1, _D), lambda i: (0, 0)),
            pl.BlockSpec((1, _D), lambda i: (0, 0)),
            pl.BlockSpec((_D, 1), lambda i: (0, 0)),
            pl.BlockSpec((1, 1), lambda i: (0, 0)),
        ],
        out_specs=[pl.BlockSpec((_BN, 1), lambda i: (i, 0))],
        out_shape=[jax.ShapeDtypeStruct((_N, 1), jnp.float32)],
    )(c, st, vst, g, be, w2, b2)[0]


def kernel(x, edge_index, params):
    src, dst = edge_index[0], edge_index[1]
    layers = params["layers"]
    outer_bn = params["outer_bn"]
    cls = params["cls"]

    hs = (x,)
    for i, lp in enumerate(layers):
        if len(hs) == 1:
            a0, a1 = _make_sc_agg_full()(hs[0], src, dst)
        else:
            a0, a1 = _make_sc_agg_split()(hs[0], hs[1], src, dst)
        eps = lp["eps"].reshape(1)
        m, st = _ka_call(eps, hs, a0, a1, lp["w1"],
                         lp["b1"].reshape(1, _D))
        g1 = lp["g1"].reshape(1, _D)
        be1 = lp["be1"].reshape(1, _D)
        b2 = lp["b2"].reshape(1, _D)
        vst = _kv_call(m, st)
        if i < len(layers) - 1:
            hh, st2 = _kb_call(m, st, vst, g1, be1, lp["w2"], b2)
            vst2 = _kv_call(hh, st2)
            ob = outer_bn[i]
            hs = _kc_call(hh, st2, vst2, ob["g"].reshape(1, _D),
                          ob["b"].reshape(1, _D))
        else:
            cc, stc = _kb2_call(m, st, vst, g1, be1, lp["w2"], b2,
                                cls["w1"], cls["b1"].reshape(1, _D))
    vstc = _kv_call(cc, stc)
    out = _ke_call(cc, stc, vstc, cls["g"].reshape(1, _D),
                   cls["be"].reshape(1, _D), cls["w2"],
                   cls["b2"].reshape(1, 1))
    return out.reshape(-1)


# speedup vs baseline: 1.0877x; 1.0877x over previous
import functools

import jax
import jax.numpy as jnp
from jax import lax
from jax.experimental import pallas as pl
from jax.experimental.pallas import tpu as pltpu
from jax.experimental.pallas import tpu_sc as plsc

_N = 10000
_E = 320000
_D = 256
_CH = 80
_ROWS = _E // _CH
_TILES = 16
_RPT = _ROWS // _TILES
_WPT = 632
_NPAD = _WPT * _TILES

_BN = 1000
_G = _N // _BN


_DH = 128


def _sc_zero_acc(zbuf, acc, base):
    def zrow(i, _):
        def zcol(j, _2):
            zbuf[i, pl.ds(j * 16, 16)] = jnp.zeros((16,), jnp.float32)
            return 0
        return lax.fori_loop(0, _DH // 16, zcol, 0)
    lax.fori_loop(0, _CH, zrow, 0)

    nfull = _WPT // _CH
    rem = _WPT - nfull * _CH
    for q in range(nfull):
        pltpu.sync_copy(zbuf, acc.at[pl.ds(base + q * _CH, _CH)])
    if rem:
        pltpu.sync_copy(zbuf.at[pl.ds(0, rem)],
                        acc.at[pl.ds(base + nfull * _CH, rem)])


_NSLOT = 4


def _sc_edge_loop(tab, ei3, rbase, didx, rows, acc, sems, nch):
    es, gs, ss = sems
    S = range(_NSLOT)

    def eload(g, b):
        pltpu.async_copy(ei3.at[rbase + g], didx[b], es[b])

    def ewait(g, b):
        pltpu.make_async_copy(ei3.at[rbase + g], didx[b], es[b]).wait()

    def gload(b):
        pltpu.async_copy(tab.at[didx[b].at[0]], rows[b], gs[b])

    def gwait(b):
        pltpu.make_async_copy(tab.at[didx[b].at[0]], rows[b], gs[b]).wait()

    def sissue(b):
        pltpu.async_copy(rows[b], acc.at[didx[b].at[1]], ss[b], add=True)

    def swait(b):
        pltpu.make_async_copy(rows[b], acc.at[didx[b].at[1]], ss[b]).wait()

    for b in S:
        eload(b, b)

    nmain = nch // _NSLOT

    def body(p, _):
        for b in S:
            ewait(_NSLOT * p + b, b)
            gload(b)
        for b in S:
            gwait(b)
            sissue(b)

        @pl.when(p + 1 < nmain)
        def _():
            for b in S:
                swait(b)
                eload(_NSLOT * (p + 1) + b, b)
        return 0

    lax.fori_loop(0, nmain, body, 0)
    for b in S:
        swait(b)
    for g in range(nmain * _NSLOT, nch):
        eload(g, 0)
        ewait(g, 0)
        gload(0)
        gwait(0)
        sissue(0)
        swait(0)


def _sc_writeback(c, base, acc, out0, out1):
    @pl.when(c == 0)
    def _():
        pltpu.sync_copy(acc.at[pl.ds(base, _WPT)], out0.at[pl.ds(base, _WPT)])

    @pl.when(c == 1)
    def _():
        pltpu.sync_copy(acc.at[pl.ds(base, _WPT)], out1.at[pl.ds(base, _WPT)])


def _sc_mesh():
    return plsc.VectorSubcoreMesh(core_axis_name="c", subcore_axis_name="s")


_EPT = _E // _TILES


@functools.cache
def _make_sc_agg_split():
    @functools.partial(
        pl.kernel,
        out_type=(jax.ShapeDtypeStruct((_NPAD, _DH), jnp.float32),
                  jax.ShapeDtypeStruct((_NPAD, _DH), jnp.float32)),
        mesh=_sc_mesh(),
        scratch_types=[
            *[pltpu.VMEM((2, _CH), jnp.int32) for _ in range(_NSLOT)],
            *[pltpu.VMEM((_CH, _DH), jnp.float32) for _ in range(_NSLOT)],
            pltpu.VMEM_SHARED((_NPAD, _DH), jnp.float32),
            *[pltpu.SemaphoreType.DMA for _ in range(3 * _NSLOT)],
        ],
    )
    def agg(h0, h1, ei3, out0, out1, *scr):
        didx = scr[:_NSLOT]
        rows = scr[_NSLOT:2 * _NSLOT]
        acc = scr[2 * _NSLOT]
        sems = (scr[2 * _NSLOT + 1:2 * _NSLOT + 1 + _NSLOT],
                scr[2 * _NSLOT + 1 + _NSLOT:2 * _NSLOT + 1 + 2 * _NSLOT],
                scr[2 * _NSLOT + 1 + 2 * _NSLOT:])
        c = lax.axis_index("c")
        s = lax.axis_index("s")
        base = s * _WPT
        _sc_zero_acc(rows[0], acc, base)
        plsc.subcore_barrier()

        rbase = s * _RPT

        @pl.when(c == 0)
        def _():
            _sc_edge_loop(h0, ei3, rbase, didx, rows, acc, sems, _RPT)

        @pl.when(c == 1)
        def _():
            _sc_edge_loop(h1, ei3, rbase, didx, rows, acc, sems, _RPT)

        plsc.subcore_barrier()
        _sc_writeback(c, base, acc, out0, out1)

    return agg


@functools.cache
def _make_sc_agg_full():
    @functools.partial(
        pl.kernel,
        out_type=(jax.ShapeDtypeStruct((_NPAD, _DH), jnp.float32),
                  jax.ShapeDtypeStruct((_NPAD, _DH), jnp.float32)),
        mesh=_sc_mesh(),
        scratch_types=[
            *[pltpu.VMEM((2, _CH), jnp.int32) for _ in range(_NSLOT)],
            *[pltpu.VMEM((_CH, _DH), jnp.float32) for _ in range(_NSLOT)],
            pltpu.VMEM_SHARED((_NPAD, _DH), jnp.float32),
            *[pltpu.SemaphoreType.DMA for _ in range(3 * _NSLOT)],
        ],
    )
    def agg(h, ei3, out0, out1, *scr):
        didx = scr[:_NSLOT]
        rows = scr[_NSLOT:2 * _NSLOT]
        acc = scr[2 * _NSLOT]
        sems = (scr[2 * _NSLOT + 1:2 * _NSLOT + 1 + _NSLOT],
                scr[2 * _NSLOT + 1 + _NSLOT:2 * _NSLOT + 1 + 2 * _NSLOT],
                scr[2 * _NSLOT + 1 + 2 * _NSLOT:])
        c = lax.axis_index("c")
        s = lax.axis_index("s")
        base = s * _WPT
        _sc_zero_acc(rows[0], acc, base)
        plsc.subcore_barrier()

        rbase = (c * _TILES + s) * (_RPT // 2)

        _sc_edge_loop(h, ei3, rbase, didx, rows, acc, sems, _RPT // 2)

        plsc.subcore_barrier()
        _sc_writeback(c, base, acc, out0, out1)

    return agg


def _lrelu(t):
    return jnp.where(t >= 0, t, 0.01 * t)


def _norm(block, st_ref, vst_ref, g_ref, b_ref):
    mu = st_ref[0:1, :] / _N
    var = vst_ref[0:1, :] / _N
    return (block - mu) / jnp.sqrt(var + 1e-5) * g_ref[...] + b_ref[...]


def _acc_stats(st_ref, v):
    @pl.when(pl.program_id(0) == 0)
    def _():
        st_ref[...] = jnp.zeros_like(st_ref)
    st_ref[0:1, :] += jnp.sum(v, axis=0, keepdims=True)


def _kv_kernel(v_ref, st_ref, vst_ref):
    mu = st_ref[0:1, :] / _N
    dv = v_ref[...] - mu
    _acc_stats(vst_ref, dv * dv)


def _kv_call(v, st):
    return pl.pallas_call(
        _kv_kernel,
        grid=(_G,),
        in_specs=[
            pl.BlockSpec((_BN, _D), lambda i: (i, 0)),
            pl.BlockSpec((8, _D), lambda i: (0, 0)),
        ],
        out_specs=[pl.BlockSpec((8, _D), lambda i: (0, 0))],
        out_shape=[jax.ShapeDtypeStruct((8, _D), jnp.float32)],
    )(v, st)[0]


def _ka_split_kernel(eps_ref, h0_ref, h1_ref, a0_ref, a1_ref, w1_ref, b1_ref,
                     m_ref, st_ref):
    h = jnp.concatenate([h0_ref[...], h1_ref[...]], axis=1)
    a = jnp.concatenate([a0_ref[...], a1_ref[...]], axis=1)
    mm = jnp.dot((1.0 + eps_ref[0]) * h + a, w1_ref[...],
                 preferred_element_type=jnp.float32) + b1_ref[...]
    m_ref[...] = mm
    _acc_stats(st_ref, mm)


def _ka_full_kernel(eps_ref, h_ref, a0_ref, a1_ref, w1_ref, b1_ref,
                    m_ref, st_ref):
    mm = jnp.dot((1.0 + eps_ref[0]) * h_ref[...] + a0_ref[...] + a1_ref[...],
                 w1_ref[...], preferred_element_type=jnp.float32) + b1_ref[...]
    m_ref[...] = mm
    _acc_stats(st_ref, mm)


def _ka_call(eps, hs, a0, a1, w1, b1):
    din = w1.shape[0]
    body = _ka_split_kernel if len(hs) == 2 else _ka_full_kernel
    h_specs = [pl.BlockSpec((_BN, h.shape[1]), lambda i: (i, 0)) for h in hs]
    return pl.pallas_call(
        body,
        grid=(_G,),
        in_specs=[
            pl.BlockSpec(memory_space=pltpu.SMEM),
            *h_specs,
            pl.BlockSpec((_BN, _DH), lambda i: (i, 0)),
            pl.BlockSpec((_BN, _DH), lambda i: (i, 0)),
            pl.BlockSpec((din, _D), lambda i: (0, 0)),
            pl.BlockSpec((1, _D), lambda i: (0, 0)),
        ],
        out_specs=[
            pl.BlockSpec((_BN, _D), lambda i: (i, 0)),
            pl.BlockSpec((8, _D), lambda i: (0, 0)),
        ],
        out_shape=[
            jax.ShapeDtypeStruct((_N, _D), jnp.float32),
            jax.ShapeDtypeStruct((8, _D), jnp.float32),
        ],
    )(eps, *hs, a0, a1, w1, b1)


def _kb_kernel(m_ref, st_ref, vst_ref, g_ref, be_ref, w2_ref, b2_ref,
               h_ref, st2_ref):
    t = _lrelu(_norm(m_ref[...], st_ref, vst_ref, g_ref, be_ref))
    hh = jnp.dot(t, w2_ref[...], preferred_element_type=jnp.float32) + b2_ref[...]
    h_ref[...] = hh
    _acc_stats(st2_ref, hh)


def _kb_call(m, st, vst, g, be, w2, b2):
    return pl.pallas_call(
        _kb_kernel,
        grid=(_G,),
        in_specs=[
            pl.BlockSpec((_BN, _D), lambda i: (i, 0)),
            pl.BlockSpec((8, _D), lambda i: (0, 0)),
            pl.BlockSpec((8, _D), lambda i: (0, 0)),
            pl.BlockSpec((1, _D), lambda i: (0, 0)),
            pl.BlockSpec((1, _D), lambda i: (0, 0)),
            pl.BlockSpec((_D, _D), lambda i: (0, 0)),
            pl.BlockSpec((1, _D), lambda i: (0, 0)),
        ],
        out_specs=[
            pl.BlockSpec((_BN, _D), lambda i: (i, 0)),
            pl.BlockSpec((8, _D), lambda i: (0, 0)),
        ],
        out_shape=[
            jax.ShapeDtypeStruct((_N, _D), jnp.float32),
            jax.ShapeDtypeStruct((8, _D), jnp.float32),
        ],
    )(m, st, vst, g, be, w2, b2)


def _kb2_kernel(m_ref, st_ref, vst_ref, g_ref, be_ref, w2_ref, b2_ref,
                cw1_ref, cb1_ref, c_ref, stc_ref):
    t = _lrelu(_norm(m_ref[...], st_ref, vst_ref, g_ref, be_ref))
    hh = jnp.dot(t, w2_ref[...], preferred_element_type=jnp.float32) + b2_ref[...]
    cc = jnp.dot(hh, cw1_ref[...], preferred_element_type=jnp.float32) + cb1_ref[...]
    c_ref[...] = cc
    _acc_stats(stc_ref, cc)


def _kb2_call(m, st, vst, g, be, w2, b2, cw1, cb1):
    return pl.pallas_call(
        _kb2_kernel,
        grid=(_G,),
        in_specs=[
            pl.BlockSpec((_BN, _D), lambda i: (i, 0)),
            pl.BlockSpec((8, _D), lambda i: (0, 0)),
            pl.BlockSpec((8, _D), lambda i: (0, 0)),
            pl.BlockSpec((1, _D), lambda i: (0, 0)),
            pl.BlockSpec((1, _D), lambda i: (0, 0)),
            pl.BlockSpec((_D, _D), lambda i: (0, 0)),
            pl.BlockSpec((1, _D), lambda i: (0, 0)),
            pl.BlockSpec((_D, _D), lambda i: (0, 0)),
            pl.BlockSpec((1, _D), lambda i: (0, 0)),
        ],
        out_specs=[
            pl.BlockSpec((_BN, _D), lambda i: (i, 0)),
            pl.BlockSpec((8, _D), lambda i: (0, 0)),
        ],
        out_shape=[
            jax.ShapeDtypeStruct((_N, _D), jnp.float32),
            jax.ShapeDtypeStruct((8, _D), jnp.float32),
        ],
    )(m, st, vst, g, be, w2, b2, cw1, cb1)


def _kc_kernel(h_ref, st_ref, vst_ref, g_ref, b_ref, o0_ref, o1_ref):
    t = _lrelu(_norm(h_ref[...], st_ref, vst_ref, g_ref, b_ref))
    o0_ref[...] = t[:, :_D // 2]
    o1_ref[...] = t[:, _D // 2:]


def _kc_call(h, st, vst, g, b):
    dh = _D // 2
    return pl.pallas_call(
        _kc_kernel,
        grid=(_G,),
        in_specs=[
            pl.BlockSpec((_BN, _D), lambda i: (i, 0)),
            pl.BlockSpec((8, _D), lambda i: (0, 0)),
            pl.BlockSpec((8, _D), lambda i: (0, 0)),
            pl.BlockSpec((1, _D), lambda i: (0, 0)),
            pl.BlockSpec((1, _D), lambda i: (0, 0)),
        ],
        out_specs=[
            pl.BlockSpec((_BN, dh), lambda i: (i, 0)),
            pl.BlockSpec((_BN, dh), lambda i: (i, 0)),
        ],
        out_shape=[
            jax.ShapeDtypeStruct((_N, dh), jnp.float32),
            jax.ShapeDtypeStruct((_N, dh), jnp.float32),
        ],
    )(h, st, vst, g, b)


def _ke_kernel(c_ref, st_ref, vst_ref, g_ref, be_ref, w2_ref, b2_ref, o_ref):
    t = _lrelu(_norm(c_ref[...], st_ref, vst_ref, g_ref, be_ref))
    o_ref[...] = jnp.dot(t, w2_ref[...],
                         preferred_element_type=jnp.float32) + b2_ref[...]


def _ke_call(c, st, vst, g, be, w2, b2):
    return pl.pallas_call(
        _ke_kernel,
        grid=(_G,),
        in_specs=[
            pl.BlockSpec((_BN, _D), lambda i: (i, 0)),
            pl.BlockSpec((8, _D), lambda i: (0, 0)),
            pl.BlockSpec((8, _D), lambda i: (0, 0)),
            pl.BlockSpec((1, _D), lambda i: (0, 0)),
            pl.BlockSpec((1, _D), lambda i: (0, 0)),
            pl.BlockSpec((_D, 1), lambda i: (0, 0)),
            pl.BlockSpec((1, 1), lambda i: (0, 0)),
        ],
        out_specs=[pl.BlockSpec((_BN, 1), lambda i: (i, 0))],
        out_shape=[jax.ShapeDtypeStruct((_N, 1), jnp.float32)],
    )(c, st, vst, g, be, w2, b2)[0]


def kernel(x, edge_index, params):
    ei3 = jnp.stack([edge_index[0].reshape(_ROWS, _CH),
                     edge_index[1].reshape(_ROWS, _CH)], axis=1)
    layers = params["layers"]
    outer_bn = params["outer_bn"]
    cls = params["cls"]

    hs = (x,)
    for i, lp in enumerate(layers):
        if len(hs) == 1:
            a0, a1 = _make_sc_agg_full()(hs[0], ei3)
        else:
            a0, a1 = _make_sc_agg_split()(hs[0], hs[1], ei3)
        eps = lp["eps"].reshape(1)
        m, st = _ka_call(eps, hs, a0, a1, lp["w1"],
                         lp["b1"].reshape(1, _D))
        g1 = lp["g1"].reshape(1, _D)
        be1 = lp["be1"].reshape(1, _D)
        b2 = lp["b2"].reshape(1, _D)
        vst = _kv_call(m, st)
        if i < len(layers) - 1:
            hh, st2 = _kb_call(m, st, vst, g1, be1, lp["w2"], b2)
            vst2 = _kv_call(hh, st2)
            ob = outer_bn[i]
            hs = _kc_call(hh, st2, vst2, ob["g"].reshape(1, _D),
                          ob["b"].reshape(1, _D))
        else:
            cc, stc = _kb2_call(m, st, vst, g1, be1, lp["w2"], b2,
                                cls["w1"], cls["b1"].reshape(1, _D))
    vstc = _kv_call(cc, stc)
    out = _ke_call(cc, stc, vstc, cls["g"].reshape(1, _D),
                   cls["be"].reshape(1, _D), cls["w2"],
                   cls["b2"].reshape(1, 1))
    return out.reshape(-1)
